# Initial kernel scaffold; baseline (speedup 1.0000x reference)
import jax
import jax.numpy as jnp
from jax.experimental import pallas as pl


def kernel(x, edge_index, W0, W1, W2, W3, b0, b1, b2, b3, sWl0, sWl1, sWl2, sWl3, sbl0, sbl1, sbl2, sbl3, sWr0, sWr1, sWr2, sWr3, Wlin, blin):
    raise NotImplementedError("write your pallas kernel here")


# speedup vs baseline: 1.3853x; 1.3853x over previous
import math

import jax
import jax.numpy as jnp
from jax.experimental import pallas as pl

H = 128


def _pad_rows(a, n_pad):
    n = a.shape[0]
    if n == n_pad:
        return a
    return jnp.pad(a, ((0, n_pad - n),) + ((0, 0),) * (a.ndim - 1))


def _mm_bias(x, W, b, relu=False, block=512):
    n, K = x.shape
    N = W.shape[1]
    n_pad = ((n + block - 1) // block) * block
    x_p = _pad_rows(x, n_pad)
    b2 = b.reshape(1, N)

    def body(x_ref, w_ref, b_ref, o_ref):
        acc = jnp.dot(x_ref[...], w_ref[...], preferred_element_type=jnp.float32)
        acc = acc + b_ref[...]
        if relu:
            acc = jnp.maximum(acc, 0.0)
        o_ref[...] = acc

    out = pl.pallas_call(
        body,
        grid=(n_pad // block,),
        in_specs=[
            pl.BlockSpec((block, K), lambda i: (i, 0)),
            pl.BlockSpec((K, N), lambda i: (0, 0)),
            pl.BlockSpec((1, N), lambda i: (0, 0)),
        ],
        out_specs=pl.BlockSpec((block, N), lambda i: (i, 0)),
        out_shape=jax.ShapeDtypeStruct((n_pad, N), jnp.float32),
    )(x_p, W, b2)
    return out[:n]


def _final_lin_logsoftmax(x, Wlin, blin, block=256):
    n, K = x.shape
    N = Wlin.shape[1]
    n_pad = ((n + block - 1) // block) * block
    x_p = _pad_rows(x, n_pad)
    b2 = blin.reshape(1, N)

    def body(x_ref, w_ref, b_ref, o_ref):
        acc = jnp.dot(x_ref[...], w_ref[...], preferred_element_type=jnp.float32)
        acc = acc + b_ref[...]
        m = jnp.max(acc, axis=1, keepdims=True)
        s = acc - m
        lse = jnp.log(jnp.sum(jnp.exp(s), axis=1, keepdims=True))
        o_ref[...] = s - lse

    out = pl.pallas_call(
        body,
        grid=(n_pad // block,),
        in_specs=[
            pl.BlockSpec((block, K), lambda i: (i, 0)),
            pl.BlockSpec((K, N), lambda i: (0, 0)),
            pl.BlockSpec((1, N), lambda i: (0, 0)),
        ],
        out_specs=pl.BlockSpec((block, N), lambda i: (i, 0)),
        out_shape=jax.ShapeDtypeStruct((n_pad, N), jnp.float32),
    )(x_p, Wlin, b2)
    return out[:n]


def _gcn_relu(x, src, dst, emask, W, b):
    n = x.shape[0]
    deg = jax.ops.segment_sum(emask, dst, num_segments=n) + 1.0
    dinv = jax.lax.rsqrt(deg)
    xW = _mm_bias(x, W, jnp.zeros_like(b))
    u = xW * dinv[:, None]
    S = jax.ops.segment_sum(u[src] * emask[:, None], dst, num_segments=n)
    out = S * dinv[:, None] + xW * (dinv * dinv)[:, None] + b[None, :]
    return jnp.maximum(out, 0.0)


def _sag_pool(x, src, dst, emask, Wl, bl, Wr):
    n = x.shape[0]
    u = (x @ Wl)[:, 0]
    r = (x @ Wr)[:, 0]
    score = jax.ops.segment_sum(u[src] * emask, dst, num_segments=n) + bl[0] + r
    k = int(math.ceil(0.5 * n))
    topv, perm = jax.lax.top_k(score, k)
    xn = x[perm] * jnp.tanh(topv)[:, None]
    mapping = jnp.full((n,), -1, dtype=src.dtype).at[perm].set(
        jnp.arange(k, dtype=src.dtype))
    nsrc = mapping[src]
    ndst = mapping[dst]
    valid = (nsrc >= 0) & (ndst >= 0) & (emask > 0.5)
    nmask = valid.astype(jnp.float32)
    nsrc = jnp.where(valid, nsrc, 0)
    ndst = jnp.where(valid, ndst, 0)
    return xn, nsrc, ndst, nmask


def kernel(x, edge_index, W0, W1, W2, W3, b0, b1, b2, b3, sWl0, sWl1, sWl2, sWl3, sbl0, sbl1, sbl2, sbl3, sWr0, sWr1, sWr2, sWr3, Wlin, blin):
    Ws = [W0, W1, W2, W3]
    bs = [b0, b1, b2, b3]
    sWls = [sWl0, sWl1, sWl2, sWl3]
    sbls = [sbl0, sbl1, sbl2, sbl3]
    sWrs = [sWr0, sWr1, sWr2, sWr3]
    src = edge_index[0]
    dst = edge_index[1]
    emask = jnp.ones((src.shape[0],), dtype=jnp.float32)
    for i in range(4):
        x = _gcn_relu(x, src, dst, emask, Ws[i], bs[i])
        x, src, dst, emask = _sag_pool(x, src, dst, emask, sWls[i], sbls[i], sWrs[i])
    return _final_lin_logsoftmax(x, Wlin, blin)


# speedup vs baseline: 7.5769x; 5.4696x over previous
import functools
import math

import jax
import jax.numpy as jnp
from jax import lax
from jax.experimental import pallas as pl
from jax.experimental.pallas import tpu as pltpu
from jax.experimental.pallas import tpu_sc as plsc

H = 128
E = 1600000
SENT = 1 << 30

EP = 1605632

_MESH = plsc.VectorSubcoreMesh(core_axis_name="c", subcore_axis_name="s")


def _rup(x, m):
    return ((x + m - 1) // m) * m


def _tc_uv(x, t, W, b, degsum):
    n_pad, K = x.shape
    blk = 256

    def body(x_ref, w_ref, b_ref, d_ref, u_ref, v_ref, *ts):
        xb = x_ref[...]
        if ts:
            xb = xb * ts[0][...]
        xW = jnp.dot(xb, w_ref[...], preferred_element_type=jnp.float32)
        dinv = lax.rsqrt(d_ref[...] + 1.0)
        u_ref[...] = xW * dinv
        v_ref[...] = xW * (dinv * dinv) + b_ref[...]

    in_specs = [
        pl.BlockSpec((blk, K), lambda i: (i, 0)),
        pl.BlockSpec((K, H), lambda i: (0, 0)),
        pl.BlockSpec((1, H), lambda i: (0, 0)),
        pl.BlockSpec((blk, 1), lambda i: (i, 0)),
    ]
    args = [x, W, b.reshape(1, H), degsum]
    if t is not None:
        in_specs.append(pl.BlockSpec((blk, 1), lambda i: (i, 0)))
        args.append(t)

    def body2(x_ref, w_ref, b_ref, d_ref, *rest):
        if t is not None:
            t_ref, u_ref, v_ref = rest
            body(x_ref, w_ref, b_ref, d_ref, u_ref, v_ref, t_ref)
        else:
            u_ref, v_ref = rest
            body(x_ref, w_ref, b_ref, d_ref, u_ref, v_ref)

    u, v = pl.pallas_call(
        body2,
        grid=(n_pad // blk,),
        in_specs=in_specs,
        out_specs=[
            pl.BlockSpec((blk, H), lambda i: (i, 0)),
            pl.BlockSpec((blk, H), lambda i: (i, 0)),
        ],
        out_shape=[
            jax.ShapeDtypeStruct((n_pad, H), jnp.float32),
            jax.ShapeDtypeStruct((n_pad, H), jnp.float32),
        ],
    )(*args)
    return u, v


def _tc_xy(S, v, degsum, Wlr, n_pad):
    blk = 256

    def body(s_ref, v_ref, d_ref, w_ref, x_ref, y_ref):
        dinv = lax.rsqrt(d_ref[...] + 1.0)
        xb = jnp.maximum(s_ref[...] * dinv + v_ref[...], 0.0)
        x_ref[...] = xb
        y_ref[...] = jnp.dot(xb, w_ref[...], preferred_element_type=jnp.float32)

    x, y2 = pl.pallas_call(
        body,
        grid=(n_pad // blk,),
        in_specs=[
            pl.BlockSpec((blk, H), lambda i: (i, 0)),
            pl.BlockSpec((blk, H), lambda i: (i, 0)),
            pl.BlockSpec((blk, 1), lambda i: (i, 0)),
            pl.BlockSpec((H, 2), lambda i: (0, 0)),
        ],
        out_specs=[
            pl.BlockSpec((blk, H), lambda i: (i, 0)),
            pl.BlockSpec((blk, 2), lambda i: (i, 0)),
        ],
        out_shape=[
            jax.ShapeDtypeStruct((n_pad, H), jnp.float32),
            jax.ShapeDtypeStruct((n_pad, 2), jnp.float32),
        ],
    )(S[:n_pad], v, degsum, Wlr)
    return x, y2


def _tc_final(g, t, Wlin, blin):
    n, K = g.shape
    N = Wlin.shape[1]
    blk = 256
    n_pad = _rup(n, blk)
    g = jnp.pad(g, ((0, n_pad - n), (0, 0)))
    t = jnp.pad(t, ((0, n_pad - n), (0, 0)))

    def body(x_ref, t_ref, w_ref, b_ref, o_ref):
        acc = jnp.dot(x_ref[...] * t_ref[...], w_ref[...],
                      preferred_element_type=jnp.float32)
        acc = acc + b_ref[...]
        m = jnp.max(acc, axis=1, keepdims=True)
        s = acc - m
        lse = jnp.log(jnp.sum(jnp.exp(s), axis=1, keepdims=True))
        o_ref[...] = s - lse

    out = pl.pallas_call(
        body,
        grid=(n_pad // blk,),
        in_specs=[
            pl.BlockSpec((blk, K), lambda i: (i, 0)),
            pl.BlockSpec((blk, 1), lambda i: (i, 0)),
            pl.BlockSpec((K, N), lambda i: (0, 0)),
            pl.BlockSpec((1, N), lambda i: (0, 0)),
        ],
        out_specs=pl.BlockSpec((blk, N), lambda i: (i, 0)),
        out_shape=jax.ShapeDtypeStruct((n_pad, N), jnp.float32),
    )(g, t, Wlin, blin.reshape(1, N))
    return out[:n]


_W1 = 1024
_NW1 = 49


def _make_edge_scalar(n, n_s, gather):
    stripe = n_s // 16
    zfull, zrem = stripe // 2048, stripe % 2048

    def body(src_hbm, dst_hbm, y_hbm, out_hbm, wsrc, wdst, ibuf, sbuf, vals,
             ones, zbuf, acc, semg):
        cid = lax.axis_index("c")
        sid = lax.axis_index("s")
        wid = sid * 2 + cid

        def init_vec(j, _):
            zbuf[pl.ds(j * 16, 16)] = jnp.zeros((16,), jnp.float32)
            return 0

        lax.fori_loop(0, 128, init_vec, 0)
        for j in range(8):
            ones[pl.ds(j * 16, 16)] = jnp.ones((16,), jnp.float32)

        base = sid * stripe
        for j in range(zfull):
            pltpu.sync_copy(zbuf, acc.at[pl.ds(base + j * 2048, 2048)])
        if zrem:
            pltpu.sync_copy(zbuf.at[pl.ds(0, zrem)],
                            acc.at[pl.ds(base + zfull * 2048, zrem)])
        plsc.subcore_barrier()

        lanes = jnp.arange(16, dtype=jnp.int32)

        def window(w, _):
            off = (wid * _NW1 + w) * _W1
            pltpu.sync_copy(dst_hbm.at[pl.ds(off, _W1)], wdst)
            if gather:
                pltpu.sync_copy(src_hbm.at[pl.ds(off, _W1)], wsrc)
            for v in range(_W1 // 16):
                d = wdst[pl.ds(v * 16, 16)]
                m = d < n
                dsel = jnp.where(m, d, n + sid)
                r, cc = v // 8, (v % 8) * 16
                ibuf[r, pl.ds(cc, 16)] = dsel
                if gather:
                    s = wsrc[pl.ds(v * 16, 16)]
                    ssel = jnp.where(m, s, sid * 16 + lanes)
                    sbuf[r, pl.ds(cc, 16)] = ssel
            if gather:
                for r in range(8):
                    pltpu.async_copy(y_hbm.at[sbuf.at[r]], vals.at[r], semg)
                for r in range(8):
                    pltpu.make_async_copy(y_hbm.at[sbuf.at[r]], vals.at[r],
                                          semg).wait()
                for r in range(8):
                    pltpu.sync_copy(vals.at[r], acc.at[ibuf.at[r]], add=True)
            else:
                for r in range(8):
                    pltpu.sync_copy(ones, acc.at[ibuf.at[r]], add=True)
            return 0

        lax.fori_loop(0, _NW1, window, 0)
        plsc.subcore_barrier()
        for j in range(zfull):
            pltpu.sync_copy(acc.at[pl.ds(base + j * 2048, 2048)], zbuf)
            pltpu.sync_copy(zbuf,
                            out_hbm.at[pl.ds(cid * n_s + base + j * 2048,
                                             2048)])
        if zrem:
            pltpu.sync_copy(acc.at[pl.ds(base + zfull * 2048, zrem)],
                            zbuf.at[pl.ds(0, zrem)])
            pltpu.sync_copy(zbuf.at[pl.ds(0, zrem)],
                            out_hbm.at[pl.ds(cid * n_s + base + zfull * 2048,
                                             zrem)])

    return pl.kernel(
        body,
        out_type=jax.ShapeDtypeStruct((2 * n_s,), jnp.float32),
        mesh=_MESH,
        scratch_types=[
            pltpu.VMEM((_W1,), jnp.int32),
            pltpu.VMEM((_W1,), jnp.int32),
            pltpu.VMEM((8, 128), jnp.int32),
            pltpu.VMEM((8, 128), jnp.int32),
            pltpu.VMEM((8, 128), jnp.float32),
            pltpu.VMEM((128,), jnp.float32),
            pltpu.VMEM((2048,), jnp.float32),
            pltpu.VMEM_SHARED((n_s,), jnp.float32),
            pltpu.SemaphoreType.DMA,
        ],
        compiler_params=pltpu.CompilerParams(needs_layout_passes=False),
    )


def _edge_scalar_deg(n, n_s, src, dst):
    k = _make_edge_scalar(n, n_s, gather=False)
    dummy_y = jnp.zeros((16,), jnp.float32)
    p = k(src, dst, dummy_y)
    return p[:n_s] + p[n_s:]


def _edge_scalar_score(n, n_s, src, dst, y):
    k = _make_edge_scalar(n, n_s, gather=True)
    p = k(src, dst, y)
    return p[:n_s] + p[n_s:]


_W2 = 2048
_NW2 = 49
_NR = 17


def _make_rowagg(n, C, chunks):
    stripe = C // 16
    zfull, zrem = stripe // 64, stripe % 64

    def body(u_hbm, src_hbm, dst_hbm, S_hbm, wsrc, wdst, wbs, wbd, ibs, ibd,
             stage, zbuf, acc, semg, sems):
        cid = lax.axis_index("c")
        sid = lax.axis_index("s")
        lanes = jnp.arange(16, dtype=jnp.int32)

        def zvec(j, _):
            zbuf[j, pl.ds(0, 16)] = jnp.zeros((16,), jnp.float32)
            return 0

        def zrow(j, _):
            for q in range(8):
                zbuf[j, pl.ds(q * 16, 16)] = jnp.zeros((16,), jnp.float32)
            return 0

        lax.fori_loop(0, 64, zrow, 0)

        def chunk_body(ci, _):
            c = ci * 2 + cid
            lo = c * C
            base = sid * stripe
            for j in range(zfull):
                pltpu.sync_copy(zbuf, acc.at[pl.ds(base + j * 64, 64)])
            if zrem:
                pltpu.sync_copy(zbuf.at[pl.ds(0, zrem)],
                                acc.at[pl.ds(base + zfull * 64, zrem)])
            plsc.subcore_barrier()

            def window(w, _):
                off = sid * (_NW2 * _W2) + w * _W2
                pltpu.sync_copy(src_hbm.at[pl.ds(off, _W2)], wsrc)
                pltpu.sync_copy(dst_hbm.at[pl.ds(off, _W2)], wdst)
                wcnt = jnp.int32(0)
                for v in range(_W2 // 16):
                    d = wdst[pl.ds(v * 16, 16)]
                    s = wsrc[pl.ds(v * 16, 16)]
                    m = (d >= lo) & (d < lo + C)
                    plsc.store_compressed(wbs.at[pl.ds(wcnt, 16)], s, mask=m)
                    plsc.store_compressed(wbd.at[pl.ds(wcnt, 16)], d - lo,
                                          mask=m)
                    wcnt = wcnt + jnp.sum(m.astype(jnp.int32))
                padbase = ((sid * _NW2 + w) * 16) % (n - 16)
                wbs[pl.ds(wcnt, 16)] = padbase + lanes
                wbd[pl.ds(wcnt, 16)] = C + lanes
                wcnt16 = (wcnt + 15) & ~15
                wfull = (wcnt16 + 127) & ~127
                for j in range(_NR * 8):
                    r, cc = j // 8, (j % 8) * 16
                    jw = j * 16

                    @pl.when(jw < wcnt16)
                    def _():
                        ibs[r, pl.ds(cc, 16)] = wbs[pl.ds(jw, 16)]
                        ibd[r, pl.ds(cc, 16)] = wbd[pl.ds(jw, 16)]

                    @pl.when((jw >= wcnt16) & (jw < wfull))
                    def _():
                        ibs[r, pl.ds(cc, 16)] = padbase + lanes
                        ibd[r, pl.ds(cc, 16)] = C + lanes

                for r in range(_NR):
                    act = r * 128 < wfull

                    if r >= 2:
                        @pl.when((r - 2) * 128 < wfull)
                        def _():
                            pltpu.make_async_copy(
                                stage.at[r % 2], acc.at[ibd.at[r - 2]],
                                sems).wait()

                    @pl.when(act)
                    def _():
                        pltpu.async_copy(u_hbm.at[ibs.at[r]], stage.at[r % 2],
                                         semg)

                    if r >= 1:
                        @pl.when((r - 1) * 128 < wfull)
                        def _():
                            pltpu.make_async_copy(
                                u_hbm.at[ibs.at[r - 1]], stage.at[(r - 1) % 2],
                                semg).wait()
                            pltpu.async_copy(stage.at[(r - 1) % 2],
                                             acc.at[ibd.at[r - 1]], sems,
                                             add=True)

                @pl.when((_NR - 1) * 128 < wfull)
                def _():
                    pltpu.make_async_copy(u_hbm.at[ibs.at[_NR - 1]],
                                          stage.at[(_NR - 1) % 2], semg).wait()
                    pltpu.async_copy(stage.at[(_NR - 1) % 2],
                                     acc.at[ibd.at[_NR - 1]], sems, add=True)
                for r in (_NR - 2, _NR - 1):
                    @pl.when(r * 128 < wfull)
                    def _():
                        pltpu.make_async_copy(stage.at[r % 2],
                                              acc.at[ibd.at[r]], sems).wait()
                return 0

            lax.fori_loop(0, _NW2, window, 0)
            plsc.subcore_barrier()
            wfull_rows, wrem_rows = stripe // 128, stripe % 128
            for j in range(wfull_rows):
                pltpu.sync_copy(acc.at[pl.ds(base + j * 128, 128)],
                                stage.at[0])
                pltpu.sync_copy(stage.at[0],
                                S_hbm.at[pl.ds(lo + base + j * 128, 128)])
            if wrem_rows:
                pltpu.sync_copy(
                    acc.at[pl.ds(base + wfull_rows * 128, wrem_rows)],
                    stage.at[0, pl.ds(0, wrem_rows)])
                pltpu.sync_copy(
                    stage.at[0, pl.ds(0, wrem_rows)],
                    S_hbm.at[pl.ds(lo + base + wfull_rows * 128, wrem_rows)])
            plsc.subcore_barrier()
            return 0

        lax.fori_loop(0, chunks // 2, chunk_body, 0)

    return pl.kernel(
        body,
        out_type=jax.ShapeDtypeStruct((chunks * C, H), jnp.float32),
        mesh=_MESH,
        scratch_types=[
            pltpu.VMEM((_W2,), jnp.int32),
            pltpu.VMEM((_W2,), jnp.int32),
            pltpu.VMEM((_NR * 128,), jnp.int32),
            pltpu.VMEM((_NR * 128,), jnp.int32),
            pltpu.VMEM((_NR, 128), jnp.int32),
            pltpu.VMEM((_NR, 128), jnp.int32),
            pltpu.VMEM((2, 128, H), jnp.float32),
            pltpu.VMEM((64, H), jnp.float32),
            pltpu.VMEM_SHARED((C + 16, H), jnp.float32),
            pltpu.SemaphoreType.DMA,
            pltpu.SemaphoreType.DMA,
        ],
        compiler_params=pltpu.CompilerParams(needs_layout_passes=False),
    )


def _make_mapping(n, k, k_pad):
    n_m = _rup(n + 2, 16 * 2048)
    rows = k_pad // (16 * 128)

    def body(perm_hbm, map_hbm, ib, vb, mb, sem):
        cid = lax.axis_index("c")
        sid = lax.axis_index("s")

        @pl.when(cid == 0)
        def _():
            def init_vec(j, _):
                mb[pl.ds(j * 16, 16)] = jnp.full((16,), -1, jnp.int32)
                return 0

            lax.fori_loop(0, 128, init_vec, 0)
            stripe = n_m // 16
            base = sid * stripe
            for j in range(stripe // 2048):
                pltpu.sync_copy(mb, map_hbm.at[pl.ds(base + j * 2048, 2048)])
            plsc.subcore_barrier()
            lanes = jnp.arange(16, dtype=jnp.int32)
            for r in range(rows):
                off = (sid * rows + r) * 128
                pltpu.sync_copy(perm_hbm.at[pl.ds(off, 128)], ib.at[0])
                for q in range(8):
                    vb[0, pl.ds(q * 16, 16)] = off + q * 16 + lanes
                pltpu.sync_copy(vb.at[0], map_hbm.at[ib.at[0]])

    return pl.kernel(
        body,
        out_type=jax.ShapeDtypeStruct((n_m,), jnp.int32),
        mesh=_MESH,
        scratch_types=[
            pltpu.VMEM((1, 128), jnp.int32),
            pltpu.VMEM((1, 128), jnp.int32),
            pltpu.VMEM((2048,), jnp.int32),
            pltpu.SemaphoreType.DMA,
        ],
        compiler_params=pltpu.CompilerParams(needs_layout_passes=False),
    )


def _make_rowgather(k_pad):
    nw = k_pad // 4096

    def body(x_hbm, perm_hbm, g_hbm, ib, stage, semg):
        cid = lax.axis_index("c")
        sid = lax.axis_index("s")
        wid = sid * 2 + cid
        for r in range(nw):
            off = (wid * nw + r) * 128
            pltpu.sync_copy(perm_hbm.at[pl.ds(off, 128)], ib.at[r % 2])
            pltpu.async_copy(x_hbm.at[ib.at[r % 2]], stage.at[r % 2], semg)
            pltpu.make_async_copy(x_hbm.at[ib.at[r % 2]], stage.at[r % 2],
                                  semg).wait()
            pltpu.sync_copy(stage.at[r % 2], g_hbm.at[pl.ds(off, 128)])

    return pl.kernel(
        body,
        out_type=jax.ShapeDtypeStruct((k_pad, H), jnp.float32),
        mesh=_MESH,
        scratch_types=[
            pltpu.VMEM((2, 128), jnp.int32),
            pltpu.VMEM((2, 128, H), jnp.float32),
            pltpu.SemaphoreType.DMA,
        ],
        compiler_params=pltpu.CompilerParams(needs_layout_passes=False),
    )


def _make_relabel(n, k, k_s):
    stripe = k_s // 16
    zfull, zrem = stripe // 2048, stripe % 2048

    def body(src_hbm, dst_hbm, map_hbm, nsrc_hbm, ndst_hbm, deg_hbm,
             wsrc, wdst, gs, gd, ob_s, ob_d, ib, ones, zbuf, acc, semg):
        cid = lax.axis_index("c")
        sid = lax.axis_index("s")
        wid = sid * 2 + cid

        def init_vec(j, _):
            zbuf[pl.ds(j * 16, 16)] = jnp.zeros((16,), jnp.float32)
            return 0

        lax.fori_loop(0, 128, init_vec, 0)
        for j in range(8):
            ones[pl.ds(j * 16, 16)] = jnp.ones((16,), jnp.float32)
        base = sid * stripe
        for j in range(zfull):
            pltpu.sync_copy(zbuf, acc.at[pl.ds(base + j * 2048, 2048)])
        if zrem:
            pltpu.sync_copy(zbuf.at[pl.ds(0, zrem)],
                            acc.at[pl.ds(base + zfull * 2048, zrem)])
        plsc.subcore_barrier()

        def window(w, _):
            off = (wid * _NW1 + w) * _W1
            pltpu.sync_copy(src_hbm.at[pl.ds(off, _W1)], wsrc)
            pltpu.sync_copy(dst_hbm.at[pl.ds(off, _W1)], wdst)
            for v in range(_W1 // 16):
                d = wdst[pl.ds(v * 16, 16)]
                s = wsrc[pl.ds(v * 16, 16)]
                r, cc = v // 8, (v % 8) * 16
                gs[r, pl.ds(cc, 16)] = s
                gd[r, pl.ds(cc, 16)] = jnp.minimum(d, n + 1)
            for r in range(8):
                pltpu.async_copy(map_hbm.at[gs.at[r]], ob_s.at[r], semg)
                pltpu.async_copy(map_hbm.at[gd.at[r]], ob_d.at[r], semg)
            for r in range(8):
                pltpu.make_async_copy(map_hbm.at[gs.at[r]], ob_s.at[r],
                                      semg).wait()
                pltpu.make_async_copy(map_hbm.at[gd.at[r]], ob_d.at[r],
                                      semg).wait()
            for v in range(_W1 // 16):
                r, cc = v // 8, (v % 8) * 16
                ns = ob_s[r, pl.ds(cc, 16)]
                nd = ob_d[r, pl.ds(cc, 16)]
                valid = (ns >= 0) & (nd >= 0)
                wsrc[pl.ds(v * 16, 16)] = jnp.where(valid, ns, 0)
                wdst[pl.ds(v * 16, 16)] = jnp.where(valid, nd, SENT)
                ib[r, pl.ds(cc, 16)] = jnp.where(valid, nd, k + sid)
            pltpu.sync_copy(wsrc, nsrc_hbm.at[pl.ds(off, _W1)])
            pltpu.sync_copy(wdst, ndst_hbm.at[pl.ds(off, _W1)])
            for r in range(8):
                pltpu.sync_copy(ones, acc.at[ib.at[r]], add=True)
            return 0

        lax.fori_loop(0, _NW1, window, 0)
        plsc.subcore_barrier()
        for j in range(zfull):
            pltpu.sync_copy(acc.at[pl.ds(base + j * 2048, 2048)], zbuf)
            pltpu.sync_copy(zbuf,
                            deg_hbm.at[pl.ds(cid * k_s + base + j * 2048,
                                             2048)])
        if zrem:
            pltpu.sync_copy(acc.at[pl.ds(base + zfull * 2048, zrem)],
                            zbuf.at[pl.ds(0, zrem)])
            pltpu.sync_copy(zbuf.at[pl.ds(0, zrem)],
                            deg_hbm.at[pl.ds(cid * k_s + base + zfull * 2048,
                                             zrem)])

    return pl.kernel(
        body,
        out_type=[
            jax.ShapeDtypeStruct((EP,), jnp.int32),
            jax.ShapeDtypeStruct((EP,), jnp.int32),
            jax.ShapeDtypeStruct((2 * k_s,), jnp.float32),
        ],
        mesh=_MESH,
        scratch_types=[
            pltpu.VMEM((_W1,), jnp.int32),
            pltpu.VMEM((_W1,), jnp.int32),
            pltpu.VMEM((8, 128), jnp.int32),
            pltpu.VMEM((8, 128), jnp.int32),
            pltpu.VMEM((8, 128), jnp.int32),
            pltpu.VMEM((8, 128), jnp.int32),
            pltpu.VMEM((8, 128), jnp.int32),
            pltpu.VMEM((128,), jnp.float32),
            pltpu.VMEM((2048,), jnp.float32),
            pltpu.VMEM_SHARED((k_s,), jnp.float32),
            pltpu.SemaphoreType.DMA,
        ],
        compiler_params=pltpu.CompilerParams(needs_layout_passes=False),
    )


def _layer_dims(n):
    n_pad = _rup(n, 256)
    chunks = 2 * max(1, math.ceil(n / (2 * 9000)))
    C = _rup(math.ceil(n / chunks), 256)
    return n_pad, chunks, C


def kernel(x, edge_index, W0, W1, W2, W3, b0, b1, b2, b3, sWl0, sWl1, sWl2, sWl3, sbl0, sbl1, sbl2, sbl3, sWr0, sWr1, sWr2, sWr3, Wlin, blin):
    Ws = [W0, W1, W2, W3]
    bs = [b0, b1, b2, b3]
    sWls = [sWl0, sWl1, sWl2, sWl3]
    sbls = [sbl0, sbl1, sbl2, sbl3]
    sWrs = [sWr0, sWr1, sWr2, sWr3]

    n = x.shape[0]
    src = jnp.pad(edge_index[0], (0, EP - E))
    dst = jnp.pad(edge_index[1], (0, EP - E), constant_values=SENT)

    n_pad, chunks, C = _layer_dims(n)
    g = jnp.pad(x, ((0, n_pad - n), (0, 0)))
    t = None

    degsum = _edge_scalar_deg(n, n_pad, src, dst).reshape(n_pad, 1)

    for i in range(4):
        u, v = _tc_uv(g, t, Ws[i], bs[i], degsum)
        S = _make_rowagg(n, C, chunks)(u, src, dst)
        Wlr = jnp.concatenate([sWls[i], sWrs[i]], axis=1)
        xx, y2 = _tc_xy(S, v, degsum, Wlr, n_pad)

        yl = y2[:, 0]
        sagg = _edge_scalar_score(n, n_pad, src, dst, yl)
        score = sagg[:n] + sbls[i][0] + y2[:n, 1]

        k = int(math.ceil(0.5 * n))
        topv, perm = lax.top_k(score, k)
        tq = jnp.tanh(topv)

        k_pad_g = _rup(k, 4096)
        perm_g = jnp.pad(perm, (0, k_pad_g - k))
        g_full = _make_rowgather(k_pad_g)(xx, perm_g)

        k_s = _rup(k, 256)
        if i < 3:
            n_m = _rup(n + 2, 16 * 2048)
            k_pad_m = _rup(k, 2048)
            perm_m = jnp.pad(perm, (0, k_pad_m - k), constant_values=n_m - 1)
            mapping = _make_mapping(n, k, k_pad_m)(perm_m)
            nsrc, ndst, degp = _make_relabel(n, k, k_s)(src, dst, mapping)
            src, dst = nsrc, ndst
            degsum = (degp[:k_s] + degp[k_s:]).reshape(k_s, 1)

        n = k
        n_pad, chunks, C = _layer_dims(n)
        g = g_full[:n_pad]
        t = jnp.pad(tq.reshape(k, 1), ((0, n_pad - n), (0, 0)))

    return _tc_final(g[:n], t[:n], Wlin, blin)


# speedup vs baseline: 7.6082x; 1.0041x over previous
import functools
import math

import jax
import jax.numpy as jnp
from jax import lax
from jax.experimental import pallas as pl
from jax.experimental.pallas import tpu as pltpu
from jax.experimental.pallas import tpu_sc as plsc

H = 128
E = 1600000
SENT = 1 << 30

EP = 1605632

_MESH = plsc.VectorSubcoreMesh(core_axis_name="c", subcore_axis_name="s")


def _rup(x, m):
    return ((x + m - 1) // m) * m


def _tc_uv(x, t, W, b, degsum):
    n_pad, K = x.shape
    blk = 256

    def body(x_ref, w_ref, b_ref, d_ref, u_ref, v_ref, *ts):
        xb = x_ref[...]
        if ts:
            xb = xb * ts[0][...]
        xW = jnp.dot(xb, w_ref[...], preferred_element_type=jnp.float32)
        dinv = lax.rsqrt(d_ref[...] + 1.0)
        u_ref[...] = xW * dinv
        v_ref[...] = xW * (dinv * dinv) + b_ref[...]

    in_specs = [
        pl.BlockSpec((blk, K), lambda i: (i, 0)),
        pl.BlockSpec((K, H), lambda i: (0, 0)),
        pl.BlockSpec((1, H), lambda i: (0, 0)),
        pl.BlockSpec((blk, 1), lambda i: (i, 0)),
    ]
    args = [x, W, b.reshape(1, H), degsum]
    if t is not None:
        in_specs.append(pl.BlockSpec((blk, 1), lambda i: (i, 0)))
        args.append(t)

    def body2(x_ref, w_ref, b_ref, d_ref, *rest):
        if t is not None:
            t_ref, u_ref, v_ref = rest
            body(x_ref, w_ref, b_ref, d_ref, u_ref, v_ref, t_ref)
        else:
            u_ref, v_ref = rest
            body(x_ref, w_ref, b_ref, d_ref, u_ref, v_ref)

    u, v = pl.pallas_call(
        body2,
        grid=(n_pad // blk,),
        in_specs=in_specs,
        out_specs=[
            pl.BlockSpec((blk, H), lambda i: (i, 0)),
            pl.BlockSpec((blk, H), lambda i: (i, 0)),
        ],
        out_shape=[
            jax.ShapeDtypeStruct((n_pad, H), jnp.float32),
            jax.ShapeDtypeStruct((n_pad, H), jnp.float32),
        ],
    )(*args)
    return u, v


def _tc_xy(S, v, degsum, Wlr, n_pad):
    blk = 256

    def body(s_ref, v_ref, d_ref, w_ref, x_ref, y_ref):
        dinv = lax.rsqrt(d_ref[...] + 1.0)
        xb = jnp.maximum(s_ref[...] * dinv + v_ref[...], 0.0)
        x_ref[...] = xb
        y_ref[...] = jnp.dot(xb, w_ref[...], preferred_element_type=jnp.float32)

    x, y2 = pl.pallas_call(
        body,
        grid=(n_pad // blk,),
        in_specs=[
            pl.BlockSpec((blk, H), lambda i: (i, 0)),
            pl.BlockSpec((blk, H), lambda i: (i, 0)),
            pl.BlockSpec((blk, 1), lambda i: (i, 0)),
            pl.BlockSpec((H, 2), lambda i: (0, 0)),
        ],
        out_specs=[
            pl.BlockSpec((blk, H), lambda i: (i, 0)),
            pl.BlockSpec((blk, 2), lambda i: (i, 0)),
        ],
        out_shape=[
            jax.ShapeDtypeStruct((n_pad, H), jnp.float32),
            jax.ShapeDtypeStruct((n_pad, 2), jnp.float32),
        ],
    )(S[:n_pad], v, degsum, Wlr)
    return x, y2


def _tc_final(g, t, Wlin, blin):
    n, K = g.shape
    N = Wlin.shape[1]
    blk = 256
    n_pad = _rup(n, blk)
    g = jnp.pad(g, ((0, n_pad - n), (0, 0)))
    t = jnp.pad(t, ((0, n_pad - n), (0, 0)))

    def body(x_ref, t_ref, w_ref, b_ref, o_ref):
        acc = jnp.dot(x_ref[...] * t_ref[...], w_ref[...],
                      preferred_element_type=jnp.float32)
        acc = acc + b_ref[...]
        m = jnp.max(acc, axis=1, keepdims=True)
        s = acc - m
        lse = jnp.log(jnp.sum(jnp.exp(s), axis=1, keepdims=True))
        o_ref[...] = s - lse

    out = pl.pallas_call(
        body,
        grid=(n_pad // blk,),
        in_specs=[
            pl.BlockSpec((blk, K), lambda i: (i, 0)),
            pl.BlockSpec((blk, 1), lambda i: (i, 0)),
            pl.BlockSpec((K, N), lambda i: (0, 0)),
            pl.BlockSpec((1, N), lambda i: (0, 0)),
        ],
        out_specs=pl.BlockSpec((blk, N), lambda i: (i, 0)),
        out_shape=jax.ShapeDtypeStruct((n_pad, N), jnp.float32),
    )(g, t, Wlin, blin.reshape(1, N))
    return out[:n]


_W1 = 1024
_NW1 = 49


def _make_edge_scalar(n, n_s, gather):
    stripe = n_s // 16
    zfull, zrem = stripe // 2048, stripe % 2048

    def body(src_hbm, dst_hbm, y_hbm, out_hbm, wsrc, wdst, ibuf, sbuf, vals,
             ones, zbuf, acc, semg, sems):
        cid = lax.axis_index("c")
        sid = lax.axis_index("s")
        wid = sid * 2 + cid

        def init_vec(j, _):
            zbuf[pl.ds(j * 16, 16)] = jnp.zeros((16,), jnp.float32)
            return 0

        lax.fori_loop(0, 128, init_vec, 0)
        for j in range(8):
            ones[pl.ds(j * 16, 16)] = jnp.ones((16,), jnp.float32)

        base = sid * stripe
        for j in range(zfull):
            pltpu.sync_copy(zbuf, acc.at[pl.ds(base + j * 2048, 2048)])
        if zrem:
            pltpu.sync_copy(zbuf.at[pl.ds(0, zrem)],
                            acc.at[pl.ds(base + zfull * 2048, zrem)])
        plsc.subcore_barrier()

        lanes = jnp.arange(16, dtype=jnp.int32)

        def wait_scatters():
            for r in range(8):
                if gather:
                    pltpu.make_async_copy(vals.at[r], acc.at[ibuf.at[r]],
                                          sems).wait()
                else:
                    pltpu.make_async_copy(ones, acc.at[ibuf.at[r]],
                                          sems).wait()

        def window(w, _):
            off = (wid * _NW1 + w) * _W1
            pltpu.sync_copy(dst_hbm.at[pl.ds(off, _W1)], wdst)
            if gather:
                pltpu.sync_copy(src_hbm.at[pl.ds(off, _W1)], wsrc)

            @pl.when(w > 0)
            def _():
                wait_scatters()

            for v in range(_W1 // 16):
                d = wdst[pl.ds(v * 16, 16)]
                m = d < n
                dsel = jnp.where(m, d, n + sid)
                r, cc = v // 8, (v % 8) * 16
                ibuf[r, pl.ds(cc, 16)] = dsel
                if gather:
                    s = wsrc[pl.ds(v * 16, 16)]
                    ssel = jnp.where(m, s, sid * 16 + lanes)
                    sbuf[r, pl.ds(cc, 16)] = ssel
            if gather:
                for r in range(8):
                    pltpu.async_copy(y_hbm.at[sbuf.at[r]], vals.at[r], semg)
                for r in range(8):
                    pltpu.make_async_copy(y_hbm.at[sbuf.at[r]], vals.at[r],
                                          semg).wait()
                for r in range(8):
                    pltpu.async_copy(vals.at[r], acc.at[ibuf.at[r]], sems,
                                     add=True)
            else:
                for r in range(8):
                    pltpu.async_copy(ones, acc.at[ibuf.at[r]], sems, add=True)
            return 0

        lax.fori_loop(0, _NW1, window, 0)
        wait_scatters()
        plsc.subcore_barrier()
        for j in range(zfull):
            pltpu.sync_copy(acc.at[pl.ds(base + j * 2048, 2048)], zbuf)
            pltpu.sync_copy(zbuf,
                            out_hbm.at[pl.ds(cid * n_s + base + j * 2048,
                                             2048)])
        if zrem:
            pltpu.sync_copy(acc.at[pl.ds(base + zfull * 2048, zrem)],
                            zbuf.at[pl.ds(0, zrem)])
            pltpu.sync_copy(zbuf.at[pl.ds(0, zrem)],
                            out_hbm.at[pl.ds(cid * n_s + base + zfull * 2048,
                                             zrem)])

    return pl.kernel(
        body,
        out_type=jax.ShapeDtypeStruct((2 * n_s,), jnp.float32),
        mesh=_MESH,
        scratch_types=[
            pltpu.VMEM((_W1,), jnp.int32),
            pltpu.VMEM((_W1,), jnp.int32),
            pltpu.VMEM((8, 128), jnp.int32),
            pltpu.VMEM((8, 128), jnp.int32),
            pltpu.VMEM((8, 128), jnp.float32),
            pltpu.VMEM((128,), jnp.float32),
            pltpu.VMEM((2048,), jnp.float32),
            pltpu.VMEM_SHARED((n_s,), jnp.float32),
            pltpu.SemaphoreType.DMA,
            pltpu.SemaphoreType.DMA,
        ],
        compiler_params=pltpu.CompilerParams(needs_layout_passes=False),
    )


def _edge_scalar_deg(n, n_s, src, dst):
    k = _make_edge_scalar(n, n_s, gather=False)
    dummy_y = jnp.zeros((16,), jnp.float32)
    p = k(src, dst, dummy_y)
    return p[:n_s] + p[n_s:]


def _edge_scalar_score(n, n_s, src, dst, y):
    k = _make_edge_scalar(n, n_s, gather=True)
    p = k(src, dst, y)
    return p[:n_s] + p[n_s:]


_W2 = 2048
_NW2 = 49
_NR = 17


def _make_rowagg(n, C, chunks):
    stripe = C // 16
    zfull, zrem = stripe // 64, stripe % 64

    def body(u_hbm, src_hbm, dst_hbm, S_hbm, wsrc, wdst, wbs, wbd, ibs, ibd,
             stage, zbuf, acc, semg, sems):
        cid = lax.axis_index("c")
        sid = lax.axis_index("s")
        lanes = jnp.arange(16, dtype=jnp.int32)

        def zvec(j, _):
            zbuf[j, pl.ds(0, 16)] = jnp.zeros((16,), jnp.float32)
            return 0

        def zrow(j, _):
            for q in range(8):
                zbuf[j, pl.ds(q * 16, 16)] = jnp.zeros((16,), jnp.float32)
            return 0

        lax.fori_loop(0, 64, zrow, 0)

        def chunk_body(ci, _):
            c = ci * 2 + cid
            lo = c * C
            base = sid * stripe
            for j in range(zfull):
                pltpu.sync_copy(zbuf, acc.at[pl.ds(base + j * 64, 64)])
            if zrem:
                pltpu.sync_copy(zbuf.at[pl.ds(0, zrem)],
                                acc.at[pl.ds(base + zfull * 64, zrem)])
            plsc.subcore_barrier()

            def window(w, _):
                off = sid * (_NW2 * _W2) + w * _W2
                pltpu.sync_copy(src_hbm.at[pl.ds(off, _W2)], wsrc)
                pltpu.sync_copy(dst_hbm.at[pl.ds(off, _W2)], wdst)
                wcnt = jnp.int32(0)
                for v in range(_W2 // 16):
                    d = wdst[pl.ds(v * 16, 16)]
                    s = wsrc[pl.ds(v * 16, 16)]
                    m = (d >= lo) & (d < lo + C)
                    plsc.store_compressed(wbs.at[pl.ds(wcnt, 16)], s, mask=m)
                    plsc.store_compressed(wbd.at[pl.ds(wcnt, 16)], d - lo,
                                          mask=m)
                    wcnt = wcnt + jnp.sum(m.astype(jnp.int32))
                padbase = ((sid * _NW2 + w) * 16) % (n - 16)
                wbs[pl.ds(wcnt, 16)] = padbase + lanes
                wbd[pl.ds(wcnt, 16)] = C + lanes
                wcnt16 = (wcnt + 15) & ~15
                wfull = (wcnt16 + 127) & ~127
                for j in range(_NR * 8):
                    r, cc = j // 8, (j % 8) * 16
                    jw = j * 16

                    @pl.when(jw < wcnt16)
                    def _():
                        ibs[r, pl.ds(cc, 16)] = wbs[pl.ds(jw, 16)]
                        ibd[r, pl.ds(cc, 16)] = wbd[pl.ds(jw, 16)]

                    @pl.when((jw >= wcnt16) & (jw < wfull))
                    def _():
                        ibs[r, pl.ds(cc, 16)] = padbase + lanes
                        ibd[r, pl.ds(cc, 16)] = C + lanes

                for r in range(_NR):
                    act = r * 128 < wfull

                    if r >= 2:
                        @pl.when((r - 2) * 128 < wfull)
                        def _():
                            pltpu.make_async_copy(
                                stage.at[r % 2], acc.at[ibd.at[r - 2]],
                                sems).wait()

                    @pl.when(act)
                    def _():
                        pltpu.async_copy(u_hbm.at[ibs.at[r]], stage.at[r % 2],
                                         semg)

                    if r >= 1:
                        @pl.when((r - 1) * 128 < wfull)
                        def _():
                            pltpu.make_async_copy(
                                u_hbm.at[ibs.at[r - 1]], stage.at[(r - 1) % 2],
                                semg).wait()
                            pltpu.async_copy(stage.at[(r - 1) % 2],
                                             acc.at[ibd.at[r - 1]], sems,
                                             add=True)

                @pl.when((_NR - 1) * 128 < wfull)
                def _():
                    pltpu.make_async_copy(u_hbm.at[ibs.at[_NR - 1]],
                                          stage.at[(_NR - 1) % 2], semg).wait()
                    pltpu.async_copy(stage.at[(_NR - 1) % 2],
                                     acc.at[ibd.at[_NR - 1]], sems, add=True)
                for r in (_NR - 2, _NR - 1):
                    @pl.when(r * 128 < wfull)
                    def _():
                        pltpu.make_async_copy(stage.at[r % 2],
                                              acc.at[ibd.at[r]], sems).wait()
                return 0

            lax.fori_loop(0, _NW2, window, 0)
            plsc.subcore_barrier()
            wfull_rows, wrem_rows = stripe // 128, stripe % 128
            for j in range(wfull_rows):
                pltpu.sync_copy(acc.at[pl.ds(base + j * 128, 128)],
                                stage.at[0])
                pltpu.sync_copy(stage.at[0],
                                S_hbm.at[pl.ds(lo + base + j * 128, 128)])
            if wrem_rows:
                pltpu.sync_copy(
                    acc.at[pl.ds(base + wfull_rows * 128, wrem_rows)],
                    stage.at[0, pl.ds(0, wrem_rows)])
                pltpu.sync_copy(
                    stage.at[0, pl.ds(0, wrem_rows)],
                    S_hbm.at[pl.ds(lo + base + wfull_rows * 128, wrem_rows)])
            plsc.subcore_barrier()
            return 0

        lax.fori_loop(0, chunks // 2, chunk_body, 0)

    return pl.kernel(
        body,
        out_type=jax.ShapeDtypeStruct((chunks * C, H), jnp.float32),
        mesh=_MESH,
        scratch_types=[
            pltpu.VMEM((_W2,), jnp.int32),
            pltpu.VMEM((_W2,), jnp.int32),
            pltpu.VMEM((_NR * 128,), jnp.int32),
            pltpu.VMEM((_NR * 128,), jnp.int32),
            pltpu.VMEM((_NR, 128), jnp.int32),
            pltpu.VMEM((_NR, 128), jnp.int32),
            pltpu.VMEM((2, 128, H), jnp.float32),
            pltpu.VMEM((64, H), jnp.float32),
            pltpu.VMEM_SHARED((C + 16, H), jnp.float32),
            pltpu.SemaphoreType.DMA,
            pltpu.SemaphoreType.DMA,
        ],
        compiler_params=pltpu.CompilerParams(needs_layout_passes=False),
    )


def _make_mapping(n, k, k_pad):
    n_m = _rup(n + 2, 16 * 2048)
    rows = k_pad // (16 * 128)

    def body(perm_hbm, map_hbm, ib, vb, mb, sem):
        cid = lax.axis_index("c")
        sid = lax.axis_index("s")

        @pl.when(cid == 0)
        def _():
            def init_vec(j, _):
                mb[pl.ds(j * 16, 16)] = jnp.full((16,), -1, jnp.int32)
                return 0

            lax.fori_loop(0, 128, init_vec, 0)
            stripe = n_m // 16
            base = sid * stripe
            for j in range(stripe // 2048):
                pltpu.sync_copy(mb, map_hbm.at[pl.ds(base + j * 2048, 2048)])
            plsc.subcore_barrier()
            lanes = jnp.arange(16, dtype=jnp.int32)
            for r in range(rows):
                off = (sid * rows + r) * 128
                pltpu.sync_copy(perm_hbm.at[pl.ds(off, 128)], ib.at[0])
                for q in range(8):
                    vb[0, pl.ds(q * 16, 16)] = off + q * 16 + lanes
                pltpu.sync_copy(vb.at[0], map_hbm.at[ib.at[0]])

    return pl.kernel(
        body,
        out_type=jax.ShapeDtypeStruct((n_m,), jnp.int32),
        mesh=_MESH,
        scratch_types=[
            pltpu.VMEM((1, 128), jnp.int32),
            pltpu.VMEM((1, 128), jnp.int32),
            pltpu.VMEM((2048,), jnp.int32),
            pltpu.SemaphoreType.DMA,
        ],
        compiler_params=pltpu.CompilerParams(needs_layout_passes=False),
    )


def _make_rowgather(k_pad):
    nw = k_pad // 4096

    def body(x_hbm, perm_hbm, g_hbm, ib, stage, semg):
        cid = lax.axis_index("c")
        sid = lax.axis_index("s")
        wid = sid * 2 + cid
        for r in range(nw):
            off = (wid * nw + r) * 128
            pltpu.sync_copy(perm_hbm.at[pl.ds(off, 128)], ib.at[r % 2])
            pltpu.async_copy(x_hbm.at[ib.at[r % 2]], stage.at[r % 2], semg)
            pltpu.make_async_copy(x_hbm.at[ib.at[r % 2]], stage.at[r % 2],
                                  semg).wait()
            pltpu.sync_copy(stage.at[r % 2], g_hbm.at[pl.ds(off, 128)])

    return pl.kernel(
        body,
        out_type=jax.ShapeDtypeStruct((k_pad, H), jnp.float32),
        mesh=_MESH,
        scratch_types=[
            pltpu.VMEM((2, 128), jnp.int32),
            pltpu.VMEM((2, 128, H), jnp.float32),
            pltpu.SemaphoreType.DMA,
        ],
        compiler_params=pltpu.CompilerParams(needs_layout_passes=False),
    )


def _make_relabel(n, k, k_s):
    stripe = k_s // 16
    zfull, zrem = stripe // 2048, stripe % 2048

    def body(src_hbm, dst_hbm, map_hbm, nsrc_hbm, ndst_hbm, deg_hbm,
             wsrc, wdst, gs, gd, ob_s, ob_d, ib, ones, zbuf, acc, semg,
             sems):
        cid = lax.axis_index("c")
        sid = lax.axis_index("s")
        wid = sid * 2 + cid

        def init_vec(j, _):
            zbuf[pl.ds(j * 16, 16)] = jnp.zeros((16,), jnp.float32)
            return 0

        lax.fori_loop(0, 128, init_vec, 0)
        for j in range(8):
            ones[pl.ds(j * 16, 16)] = jnp.ones((16,), jnp.float32)
        base = sid * stripe
        for j in range(zfull):
            pltpu.sync_copy(zbuf, acc.at[pl.ds(base + j * 2048, 2048)])
        if zrem:
            pltpu.sync_copy(zbuf.at[pl.ds(0, zrem)],
                            acc.at[pl.ds(base + zfull * 2048, zrem)])
        plsc.subcore_barrier()

        def wait_scatters():
            for r in range(8):
                pltpu.make_async_copy(ones, acc.at[ib.at[r]], sems).wait()

        def window(w, _):
            off = (wid * _NW1 + w) * _W1
            pltpu.sync_copy(src_hbm.at[pl.ds(off, _W1)], wsrc)
            pltpu.sync_copy(dst_hbm.at[pl.ds(off, _W1)], wdst)
            for v in range(_W1 // 16):
                d = wdst[pl.ds(v * 16, 16)]
                s = wsrc[pl.ds(v * 16, 16)]
                r, cc = v // 8, (v % 8) * 16
                gs[r, pl.ds(cc, 16)] = s
                gd[r, pl.ds(cc, 16)] = jnp.minimum(d, n + 1)
            for r in range(8):
                pltpu.async_copy(map_hbm.at[gs.at[r]], ob_s.at[r], semg)
                pltpu.async_copy(map_hbm.at[gd.at[r]], ob_d.at[r], semg)
            for r in range(8):
                pltpu.make_async_copy(map_hbm.at[gs.at[r]], ob_s.at[r],
                                      semg).wait()
                pltpu.make_async_copy(map_hbm.at[gd.at[r]], ob_d.at[r],
                                      semg).wait()

            @pl.when(w > 0)
            def _():
                wait_scatters()

            for v in range(_W1 // 16):
                r, cc = v // 8, (v % 8) * 16
                ns = ob_s[r, pl.ds(cc, 16)]
                nd = ob_d[r, pl.ds(cc, 16)]
                valid = (ns >= 0) & (nd >= 0)
                wsrc[pl.ds(v * 16, 16)] = jnp.where(valid, ns, 0)
                wdst[pl.ds(v * 16, 16)] = jnp.where(valid, nd, SENT)
                ib[r, pl.ds(cc, 16)] = jnp.where(valid, nd, k + sid)
            pltpu.sync_copy(wsrc, nsrc_hbm.at[pl.ds(off, _W1)])
            pltpu.sync_copy(wdst, ndst_hbm.at[pl.ds(off, _W1)])
            for r in range(8):
                pltpu.async_copy(ones, acc.at[ib.at[r]], sems, add=True)
            return 0

        lax.fori_loop(0, _NW1, window, 0)
        wait_scatters()
        plsc.subcore_barrier()
        for j in range(zfull):
            pltpu.sync_copy(acc.at[pl.ds(base + j * 2048, 2048)], zbuf)
            pltpu.sync_copy(zbuf,
                            deg_hbm.at[pl.ds(cid * k_s + base + j * 2048,
                                             2048)])
        if zrem:
            pltpu.sync_copy(acc.at[pl.ds(base + zfull * 2048, zrem)],
                            zbuf.at[pl.ds(0, zrem)])
            pltpu.sync_copy(zbuf.at[pl.ds(0, zrem)],
                            deg_hbm.at[pl.ds(cid * k_s + base + zfull * 2048,
                                             zrem)])

    return pl.kernel(
        body,
        out_type=[
            jax.ShapeDtypeStruct((EP,), jnp.int32),
            jax.ShapeDtypeStruct((EP,), jnp.int32),
            jax.ShapeDtypeStruct((2 * k_s,), jnp.float32),
        ],
        mesh=_MESH,
        scratch_types=[
            pltpu.VMEM((_W1,), jnp.int32),
            pltpu.VMEM((_W1,), jnp.int32),
            pltpu.VMEM((8, 128), jnp.int32),
            pltpu.VMEM((8, 128), jnp.int32),
            pltpu.VMEM((8, 128), jnp.int32),
            pltpu.VMEM((8, 128), jnp.int32),
            pltpu.VMEM((8, 128), jnp.int32),
            pltpu.VMEM((128,), jnp.float32),
            pltpu.VMEM((2048,), jnp.float32),
            pltpu.VMEM_SHARED((k_s,), jnp.float32),
            pltpu.SemaphoreType.DMA,
            pltpu.SemaphoreType.DMA,
        ],
        compiler_params=pltpu.CompilerParams(needs_layout_passes=False),
    )


def _layer_dims(n):
    n_pad = _rup(n, 256)
    chunks = 2 * max(1, math.ceil(n / (2 * 9000)))
    C = _rup(math.ceil(n / chunks), 256)
    return n_pad, chunks, C


def kernel(x, edge_index, W0, W1, W2, W3, b0, b1, b2, b3, sWl0, sWl1, sWl2, sWl3, sbl0, sbl1, sbl2, sbl3, sWr0, sWr1, sWr2, sWr3, Wlin, blin):
    Ws = [W0, W1, W2, W3]
    bs = [b0, b1, b2, b3]
    sWls = [sWl0, sWl1, sWl2, sWl3]
    sbls = [sbl0, sbl1, sbl2, sbl3]
    sWrs = [sWr0, sWr1, sWr2, sWr3]

    n = x.shape[0]
    src = jnp.pad(edge_index[0], (0, EP - E))
    dst = jnp.pad(edge_index[1], (0, EP - E), constant_values=SENT)

    n_pad, chunks, C = _layer_dims(n)
    g = jnp.pad(x, ((0, n_pad - n), (0, 0)))
    t = None

    degsum = _edge_scalar_deg(n, n_pad, src, dst).reshape(n_pad, 1)

    for i in range(4):
        u, v = _tc_uv(g, t, Ws[i], bs[i], degsum)
        S = _make_rowagg(n, C, chunks)(u, src, dst)
        Wlr = jnp.concatenate([sWls[i], sWrs[i]], axis=1)
        xx, y2 = _tc_xy(S, v, degsum, Wlr, n_pad)

        yl = y2[:, 0]
        sagg = _edge_scalar_score(n, n_pad, src, dst, yl)
        score = sagg[:n] + sbls[i][0] + y2[:n, 1]

        k = int(math.ceil(0.5 * n))
        topv, perm = lax.top_k(score, k)
        tq = jnp.tanh(topv)

        k_pad_g = _rup(k, 4096)
        perm_g = jnp.pad(perm, (0, k_pad_g - k))
        g_full = _make_rowgather(k_pad_g)(xx, perm_g)

        k_s = _rup(k, 256)
        if i < 3:
            n_m = _rup(n + 2, 16 * 2048)
            k_pad_m = _rup(k, 2048)
            perm_m = jnp.pad(perm, (0, k_pad_m - k), constant_values=n_m - 1)
            mapping = _make_mapping(n, k, k_pad_m)(perm_m)
            nsrc, ndst, degp = _make_relabel(n, k, k_s)(src, dst, mapping)
            src, dst = nsrc, ndst
            degsum = (degp[:k_s] + degp[k_s:]).reshape(k_s, 1)

        n = k
        n_pad, chunks, C = _layer_dims(n)
        g = g_full[:n_pad]
        t = jnp.pad(tq.reshape(k, 1), ((0, n_pad - n), (0, 0)))

    return _tc_final(g[:n], t[:n], Wlin, blin)


# speedup vs baseline: 12.9471x; 1.7017x over previous
import functools
import math

import jax
import jax.numpy as jnp
from jax import lax
from jax.experimental import pallas as pl
from jax.experimental.pallas import tpu as pltpu
from jax.experimental.pallas import tpu_sc as plsc

H = 128
E = 1600000
SENT = 1 << 30

EP = 1605632

_MESH = plsc.VectorSubcoreMesh(core_axis_name="c", subcore_axis_name="s")


def _rup(x, m):
    return ((x + m - 1) // m) * m


def _tc_uv(x, t, W, b, degsum):
    n_pad, K = x.shape
    blk = 256

    def body(x_ref, w_ref, b_ref, d_ref, u_ref, v_ref, *ts):
        xb = x_ref[...]
        if ts:
            xb = xb * ts[0][...]
        xW = jnp.dot(xb, w_ref[...], preferred_element_type=jnp.float32)
        dinv = lax.rsqrt(d_ref[...] + 1.0)
        u_ref[...] = xW * dinv
        v_ref[...] = xW * (dinv * dinv) + b_ref[...]

    in_specs = [
        pl.BlockSpec((blk, K), lambda i: (i, 0)),
        pl.BlockSpec((K, H), lambda i: (0, 0)),
        pl.BlockSpec((1, H), lambda i: (0, 0)),
        pl.BlockSpec((blk, 1), lambda i: (i, 0)),
    ]
    args = [x, W, b.reshape(1, H), degsum]
    if t is not None:
        in_specs.append(pl.BlockSpec((blk, 1), lambda i: (i, 0)))
        args.append(t)

    def body2(x_ref, w_ref, b_ref, d_ref, *rest):
        if t is not None:
            t_ref, u_ref, v_ref = rest
            body(x_ref, w_ref, b_ref, d_ref, u_ref, v_ref, t_ref)
        else:
            u_ref, v_ref = rest
            body(x_ref, w_ref, b_ref, d_ref, u_ref, v_ref)

    u, v = pl.pallas_call(
        body2,
        grid=(n_pad // blk,),
        in_specs=in_specs,
        out_specs=[
            pl.BlockSpec((blk, H), lambda i: (i, 0)),
            pl.BlockSpec((blk, H), lambda i: (i, 0)),
        ],
        out_shape=[
            jax.ShapeDtypeStruct((n_pad, H), jnp.float32),
            jax.ShapeDtypeStruct((n_pad, H), jnp.float32),
        ],
    )(*args)
    return u, v


def _tc_xy(S, v, degsum, Wlr, n_pad):
    blk = 256

    def body(s_ref, v_ref, d_ref, w_ref, x_ref, y_ref):
        dinv = lax.rsqrt(d_ref[...] + 1.0)
        xb = jnp.maximum(s_ref[...] * dinv + v_ref[...], 0.0)
        x_ref[...] = xb
        y_ref[...] = jnp.dot(xb, w_ref[...], preferred_element_type=jnp.float32)

    x, y2 = pl.pallas_call(
        body,
        grid=(n_pad // blk,),
        in_specs=[
            pl.BlockSpec((blk, H), lambda i: (i, 0)),
            pl.BlockSpec((blk, H), lambda i: (i, 0)),
            pl.BlockSpec((blk, 1), lambda i: (i, 0)),
            pl.BlockSpec((H, 2), lambda i: (0, 0)),
        ],
        out_specs=[
            pl.BlockSpec((blk, H), lambda i: (i, 0)),
            pl.BlockSpec((blk, 2), lambda i: (i, 0)),
        ],
        out_shape=[
            jax.ShapeDtypeStruct((n_pad, H), jnp.float32),
            jax.ShapeDtypeStruct((n_pad, 2), jnp.float32),
        ],
    )(S[:n_pad], v, degsum, Wlr)
    return x, y2


def _tc_final(g, t, Wlin, blin):
    n, K = g.shape
    N = Wlin.shape[1]
    blk = 256
    n_pad = _rup(n, blk)
    g = jnp.pad(g, ((0, n_pad - n), (0, 0)))
    t = jnp.pad(t, ((0, n_pad - n), (0, 0)))

    def body(x_ref, t_ref, w_ref, b_ref, o_ref):
        acc = jnp.dot(x_ref[...] * t_ref[...], w_ref[...],
                      preferred_element_type=jnp.float32)
        acc = acc + b_ref[...]
        m = jnp.max(acc, axis=1, keepdims=True)
        s = acc - m
        lse = jnp.log(jnp.sum(jnp.exp(s), axis=1, keepdims=True))
        o_ref[...] = s - lse

    out = pl.pallas_call(
        body,
        grid=(n_pad // blk,),
        in_specs=[
            pl.BlockSpec((blk, K), lambda i: (i, 0)),
            pl.BlockSpec((blk, 1), lambda i: (i, 0)),
            pl.BlockSpec((K, N), lambda i: (0, 0)),
            pl.BlockSpec((1, N), lambda i: (0, 0)),
        ],
        out_specs=pl.BlockSpec((blk, N), lambda i: (i, 0)),
        out_shape=jax.ShapeDtypeStruct((n_pad, N), jnp.float32),
    )(g, t, Wlin, blin.reshape(1, N))
    return out[:n]


_W1 = 1024
_NW1 = 49


def _make_edge_scalar(n, n_s, gather):
    stripe = n_s // 16
    zfull, zrem = stripe // 2048, stripe % 2048

    def body(src_hbm, dst_hbm, y_hbm, out_hbm, wsrc, wdst, ibuf, sbuf, vals,
             ones, zbuf, acc, semg, sems):
        cid = lax.axis_index("c")
        sid = lax.axis_index("s")
        wid = sid * 2 + cid

        def init_vec(j, _):
            zbuf[pl.ds(j * 16, 16)] = jnp.zeros((16,), jnp.float32)
            return 0

        lax.fori_loop(0, 128, init_vec, 0)
        for j in range(8):
            ones[pl.ds(j * 16, 16)] = jnp.ones((16,), jnp.float32)

        base = sid * stripe
        for j in range(zfull):
            pltpu.sync_copy(zbuf, acc.at[pl.ds(base + j * 2048, 2048)])
        if zrem:
            pltpu.sync_copy(zbuf.at[pl.ds(0, zrem)],
                            acc.at[pl.ds(base + zfull * 2048, zrem)])
        plsc.subcore_barrier()

        lanes = jnp.arange(16, dtype=jnp.int32)

        def wait_scatters():
            for r in range(8):
                if gather:
                    pltpu.make_async_copy(vals.at[r], acc.at[ibuf.at[r]],
                                          sems).wait()
                else:
                    pltpu.make_async_copy(ones, acc.at[ibuf.at[r]],
                                          sems).wait()

        def window(w, _):
            off = (wid * _NW1 + w) * _W1
            pltpu.sync_copy(dst_hbm.at[pl.ds(off, _W1)], wdst)
            if gather:
                pltpu.sync_copy(src_hbm.at[pl.ds(off, _W1)], wsrc)

            @pl.when(w > 0)
            def _():
                wait_scatters()

            for v in range(_W1 // 16):
                d = wdst[pl.ds(v * 16, 16)]
                m = d < n
                dsel = jnp.where(m, d, n + sid)
                r, cc = v // 8, (v % 8) * 16
                ibuf[r, pl.ds(cc, 16)] = dsel
                if gather:
                    s = wsrc[pl.ds(v * 16, 16)]
                    ssel = jnp.where(m, s, sid * 16 + lanes)
                    sbuf[r, pl.ds(cc, 16)] = ssel
            if gather:
                for r in range(8):
                    pltpu.async_copy(y_hbm.at[sbuf.at[r]], vals.at[r], semg)
                for r in range(8):
                    pltpu.make_async_copy(y_hbm.at[sbuf.at[r]], vals.at[r],
                                          semg).wait()
                for r in range(8):
                    pltpu.async_copy(vals.at[r], acc.at[ibuf.at[r]], sems,
                                     add=True)
            else:
                for r in range(8):
                    pltpu.async_copy(ones, acc.at[ibuf.at[r]], sems, add=True)
            return 0

        lax.fori_loop(0, _NW1, window, 0)
        wait_scatters()
        plsc.subcore_barrier()
        for j in range(zfull):
            pltpu.sync_copy(acc.at[pl.ds(base + j * 2048, 2048)], zbuf)
            pltpu.sync_copy(zbuf,
                            out_hbm.at[pl.ds(cid * n_s + base + j * 2048,
                                             2048)])
        if zrem:
            pltpu.sync_copy(acc.at[pl.ds(base + zfull * 2048, zrem)],
                            zbuf.at[pl.ds(0, zrem)])
            pltpu.sync_copy(zbuf.at[pl.ds(0, zrem)],
                            out_hbm.at[pl.ds(cid * n_s + base + zfull * 2048,
                                             zrem)])

    return pl.kernel(
        body,
        out_type=jax.ShapeDtypeStruct((2 * n_s,), jnp.float32),
        mesh=_MESH,
        scratch_types=[
            pltpu.VMEM((_W1,), jnp.int32),
            pltpu.VMEM((_W1,), jnp.int32),
            pltpu.VMEM((8, 128), jnp.int32),
            pltpu.VMEM((8, 128), jnp.int32),
            pltpu.VMEM((8, 128), jnp.float32),
            pltpu.VMEM((128,), jnp.float32),
            pltpu.VMEM((2048,), jnp.float32),
            pltpu.VMEM_SHARED((n_s,), jnp.float32),
            pltpu.SemaphoreType.DMA,
            pltpu.SemaphoreType.DMA,
        ],
        compiler_params=pltpu.CompilerParams(needs_layout_passes=False),
    )


def _edge_scalar_deg(n, n_s, src, dst):
    k = _make_edge_scalar(n, n_s, gather=False)
    dummy_y = jnp.zeros((16,), jnp.float32)
    p = k(src, dst, dummy_y)
    return p[:n_s] + p[n_s:]


def _edge_scalar_score(n, n_s, src, dst, y):
    k = _make_edge_scalar(n, n_s, gather=True)
    p = k(src, dst, y)
    return p[:n_s] + p[n_s:]


_W2 = 2048
_NW2 = 49
_NR = 17


def _make_rowagg(n, C, chunks):
    stripe = C // 16
    zfull, zrem = stripe // 64, stripe % 64

    def body(u_hbm, src_hbm, dst_hbm, S_hbm, wsrc, wdst, wbs, wbd, ibs, ibd,
             stage, zbuf, acc, semg, sems):
        cid = lax.axis_index("c")
        sid = lax.axis_index("s")
        lanes = jnp.arange(16, dtype=jnp.int32)

        def zvec(j, _):
            zbuf[j, pl.ds(0, 16)] = jnp.zeros((16,), jnp.float32)
            return 0

        def zrow(j, _):
            for q in range(8):
                zbuf[j, pl.ds(q * 16, 16)] = jnp.zeros((16,), jnp.float32)
            return 0

        lax.fori_loop(0, 64, zrow, 0)

        def chunk_body(ci, _):
            c = ci * 2 + cid
            lo = c * C
            base = sid * stripe
            for j in range(zfull):
                pltpu.sync_copy(zbuf, acc.at[pl.ds(base + j * 64, 64)])
            if zrem:
                pltpu.sync_copy(zbuf.at[pl.ds(0, zrem)],
                                acc.at[pl.ds(base + zfull * 64, zrem)])
            plsc.subcore_barrier()

            def window(w, _):
                off = sid * (_NW2 * _W2) + w * _W2
                pltpu.sync_copy(src_hbm.at[pl.ds(off, _W2)], wsrc)
                pltpu.sync_copy(dst_hbm.at[pl.ds(off, _W2)], wdst)
                wcnt = jnp.int32(0)
                for v in range(_W2 // 16):
                    d = wdst[pl.ds(v * 16, 16)]
                    s = wsrc[pl.ds(v * 16, 16)]
                    m = (d >= lo) & (d < lo + C)
                    plsc.store_compressed(wbs.at[pl.ds(wcnt, 16)], s, mask=m)
                    plsc.store_compressed(wbd.at[pl.ds(wcnt, 16)], d - lo,
                                          mask=m)
                    wcnt = wcnt + jnp.sum(m.astype(jnp.int32))
                padbase = ((sid * _NW2 + w) * 16) % (n - 16)
                wbs[pl.ds(wcnt, 16)] = padbase + lanes
                wbd[pl.ds(wcnt, 16)] = C + lanes
                wcnt16 = (wcnt + 15) & ~15
                wfull = (wcnt16 + 127) & ~127
                for j in range(_NR * 8):
                    r, cc = j // 8, (j % 8) * 16
                    jw = j * 16

                    @pl.when(jw < wcnt16)
                    def _():
                        ibs[r, pl.ds(cc, 16)] = wbs[pl.ds(jw, 16)]
                        ibd[r, pl.ds(cc, 16)] = wbd[pl.ds(jw, 16)]

                    @pl.when((jw >= wcnt16) & (jw < wfull))
                    def _():
                        ibs[r, pl.ds(cc, 16)] = padbase + lanes
                        ibd[r, pl.ds(cc, 16)] = C + lanes

                for r in range(_NR):
                    act = r * 128 < wfull

                    if r >= 2:
                        @pl.when((r - 2) * 128 < wfull)
                        def _():
                            pltpu.make_async_copy(
                                stage.at[r % 2], acc.at[ibd.at[r - 2]],
                                sems).wait()

                    @pl.when(act)
                    def _():
                        pltpu.async_copy(u_hbm.at[ibs.at[r]], stage.at[r % 2],
                                         semg)

                    if r >= 1:
                        @pl.when((r - 1) * 128 < wfull)
                        def _():
                            pltpu.make_async_copy(
                                u_hbm.at[ibs.at[r - 1]], stage.at[(r - 1) % 2],
                                semg).wait()
                            pltpu.async_copy(stage.at[(r - 1) % 2],
                                             acc.at[ibd.at[r - 1]], sems,
                                             add=True)

                @pl.when((_NR - 1) * 128 < wfull)
                def _():
                    pltpu.make_async_copy(u_hbm.at[ibs.at[_NR - 1]],
                                          stage.at[(_NR - 1) % 2], semg).wait()
                    pltpu.async_copy(stage.at[(_NR - 1) % 2],
                                     acc.at[ibd.at[_NR - 1]], sems, add=True)
                for r in (_NR - 2, _NR - 1):
                    @pl.when(r * 128 < wfull)
                    def _():
                        pltpu.make_async_copy(stage.at[r % 2],
                                              acc.at[ibd.at[r]], sems).wait()
                return 0

            lax.fori_loop(0, _NW2, window, 0)
            plsc.subcore_barrier()
            wfull_rows, wrem_rows = stripe // 128, stripe % 128
            for j in range(wfull_rows):
                pltpu.sync_copy(acc.at[pl.ds(base + j * 128, 128)],
                                stage.at[0])
                pltpu.sync_copy(stage.at[0],
                                S_hbm.at[pl.ds(lo + base + j * 128, 128)])
            if wrem_rows:
                pltpu.sync_copy(
                    acc.at[pl.ds(base + wfull_rows * 128, wrem_rows)],
                    stage.at[0, pl.ds(0, wrem_rows)])
                pltpu.sync_copy(
                    stage.at[0, pl.ds(0, wrem_rows)],
                    S_hbm.at[pl.ds(lo + base + wfull_rows * 128, wrem_rows)])
            plsc.subcore_barrier()
            return 0

        lax.fori_loop(0, chunks // 2, chunk_body, 0)

    return pl.kernel(
        body,
        out_type=jax.ShapeDtypeStruct((chunks * C, H), jnp.float32),
        mesh=_MESH,
        scratch_types=[
            pltpu.VMEM((_W2,), jnp.int32),
            pltpu.VMEM((_W2,), jnp.int32),
            pltpu.VMEM((_NR * 128,), jnp.int32),
            pltpu.VMEM((_NR * 128,), jnp.int32),
            pltpu.VMEM((_NR, 128), jnp.int32),
            pltpu.VMEM((_NR, 128), jnp.int32),
            pltpu.VMEM((2, 128, H), jnp.float32),
            pltpu.VMEM((64, H), jnp.float32),
            pltpu.VMEM_SHARED((C + 16, H), jnp.float32),
            pltpu.SemaphoreType.DMA,
            pltpu.SemaphoreType.DMA,
        ],
        compiler_params=pltpu.CompilerParams(needs_layout_passes=False),
    )


def _make_deg2(n, n_s):
    zn = n_s // 16

    def body(dst_hbm, out_hbm, wdst, acc, sem):
        cid = lax.axis_index("c")
        sid = lax.axis_index("s")
        wid = sid * 2 + cid
        ones16 = jnp.ones((16,), jnp.float32)

        def zvec(j, _):
            acc[pl.ds(j * 16, 16)] = jnp.zeros((16,), jnp.float32)
            return 0

        lax.fori_loop(0, zn, zvec, 0)

        def window(w, _):
            off = (wid * _NW1 + w) * _W1
            pltpu.sync_copy(dst_hbm.at[pl.ds(off, _W1)], wdst)
            for v in range(_W1 // 16):
                d = wdst[pl.ds(v * 16, 16)]
                m = d < n
                dsel = jnp.where(m, d, n)
                plsc.addupdate_scatter(acc, [dsel], ones16)
            return 0

        lax.fori_loop(0, _NW1, window, 0)
        base = wid * n_s
        for j in range(n_s // 2048):
            pltpu.sync_copy(acc.at[pl.ds(j * 2048, 2048)],
                            out_hbm.at[pl.ds(base + j * 2048, 2048)])
        rem = n_s % 2048
        if rem:
            pltpu.sync_copy(acc.at[pl.ds(n_s - rem, rem)],
                            out_hbm.at[pl.ds(base + n_s - rem, rem)])

    return pl.kernel(
        body,
        out_type=jax.ShapeDtypeStruct((32 * n_s,), jnp.float32),
        mesh=_MESH,
        scratch_types=[
            pltpu.VMEM((_W1,), jnp.int32),
            pltpu.VMEM((n_s,), jnp.float32),
            pltpu.SemaphoreType.DMA,
        ],
        compiler_params=pltpu.CompilerParams(needs_layout_passes=False),
    )


def _deg2(n, n_s, dst):
    p = _make_deg2(n, n_s)(dst)
    return jnp.sum(p.reshape(32, n_s), axis=0)[:n_s]


def _make_mapping(n, k, k_pad):
    n_m = _rup(n + 2, 16 * 2048)
    rows = k_pad // (16 * 128)

    def body(perm_hbm, map_hbm, ib, vb, mb, sem):
        cid = lax.axis_index("c")
        sid = lax.axis_index("s")

        @pl.when(cid == 0)
        def _():
            def init_vec(j, _):
                mb[pl.ds(j * 16, 16)] = jnp.full((16,), -1, jnp.int32)
                return 0

            lax.fori_loop(0, 128, init_vec, 0)
            stripe = n_m // 16
            base = sid * stripe
            for j in range(stripe // 2048):
                pltpu.sync_copy(mb, map_hbm.at[pl.ds(base + j * 2048, 2048)])
            plsc.subcore_barrier()
            lanes = jnp.arange(16, dtype=jnp.int32)
            for r in range(rows):
                off = (sid * rows + r) * 128
                pltpu.sync_copy(perm_hbm.at[pl.ds(off, 128)], ib.at[0])
                for q in range(8):
                    vb[0, pl.ds(q * 16, 16)] = off + q * 16 + lanes
                pltpu.sync_copy(vb.at[0], map_hbm.at[ib.at[0]])

    return pl.kernel(
        body,
        out_type=jax.ShapeDtypeStruct((n_m,), jnp.int32),
        mesh=_MESH,
        scratch_types=[
            pltpu.VMEM((1, 128), jnp.int32),
            pltpu.VMEM((1, 128), jnp.int32),
            pltpu.VMEM((2048,), jnp.int32),
            pltpu.SemaphoreType.DMA,
        ],
        compiler_params=pltpu.CompilerParams(needs_layout_passes=False),
    )


def _make_rowgather(k_pad):
    nw = k_pad // 4096

    def body(x_hbm, perm_hbm, g_hbm, ib, stage, semg):
        cid = lax.axis_index("c")
        sid = lax.axis_index("s")
        wid = sid * 2 + cid
        for r in range(nw):
            off = (wid * nw + r) * 128
            pltpu.sync_copy(perm_hbm.at[pl.ds(off, 128)], ib.at[r % 2])
            pltpu.async_copy(x_hbm.at[ib.at[r % 2]], stage.at[r % 2], semg)
            pltpu.make_async_copy(x_hbm.at[ib.at[r % 2]], stage.at[r % 2],
                                  semg).wait()
            pltpu.sync_copy(stage.at[r % 2], g_hbm.at[pl.ds(off, 128)])

    return pl.kernel(
        body,
        out_type=jax.ShapeDtypeStruct((k_pad, H), jnp.float32),
        mesh=_MESH,
        scratch_types=[
            pltpu.VMEM((2, 128), jnp.int32),
            pltpu.VMEM((2, 128, H), jnp.float32),
            pltpu.SemaphoreType.DMA,
        ],
        compiler_params=pltpu.CompilerParams(needs_layout_passes=False),
    )


def _make_relabel2(n, n_m):
    n_m2 = _rup(n + 2, 2048)

    def body(src_hbm, dst_hbm, map_hbm, nsrc_hbm, ndst_hbm,
             wsrc, wdst, mapv, sem):
        cid = lax.axis_index("c")
        sid = lax.axis_index("s")
        wid = sid * 2 + cid
        for j in range(n_m2 // 2048):
            pltpu.sync_copy(map_hbm.at[pl.ds(j * 2048, 2048)],
                            mapv.at[pl.ds(j * 2048, 2048)])

        def window(w, _):
            off = (wid * _NW1 + w) * _W1
            pltpu.sync_copy(src_hbm.at[pl.ds(off, _W1)], wsrc)
            pltpu.sync_copy(dst_hbm.at[pl.ds(off, _W1)], wdst)
            for v in range(_W1 // 16):
                d = wdst[pl.ds(v * 16, 16)]
                s = wsrc[pl.ds(v * 16, 16)]
                ns = plsc.load_gather(mapv, [s])
                nd = plsc.load_gather(mapv, [jnp.minimum(d, n + 1)])
                valid = (ns >= 0) & (nd >= 0)
                wsrc[pl.ds(v * 16, 16)] = jnp.where(valid, ns, 0)
                wdst[pl.ds(v * 16, 16)] = jnp.where(valid, nd, SENT)
            pltpu.sync_copy(wsrc, nsrc_hbm.at[pl.ds(off, _W1)])
            pltpu.sync_copy(wdst, ndst_hbm.at[pl.ds(off, _W1)])
            return 0

        lax.fori_loop(0, _NW1, window, 0)

    return pl.kernel(
        body,
        out_type=[
            jax.ShapeDtypeStruct((EP,), jnp.int32),
            jax.ShapeDtypeStruct((EP,), jnp.int32),
        ],
        mesh=_MESH,
        scratch_types=[
            pltpu.VMEM((_W1,), jnp.int32),
            pltpu.VMEM((_W1,), jnp.int32),
            pltpu.VMEM((n_m2,), jnp.int32),
            pltpu.SemaphoreType.DMA,
        ],
        compiler_params=pltpu.CompilerParams(needs_layout_passes=False),
    )


def _layer_dims(n):
    n_pad = _rup(n, 256)
    chunks = 2 * max(1, math.ceil(n / (2 * 9000)))
    C = _rup(math.ceil(n / chunks), 256)
    return n_pad, chunks, C


def kernel(x, edge_index, W0, W1, W2, W3, b0, b1, b2, b3, sWl0, sWl1, sWl2, sWl3, sbl0, sbl1, sbl2, sbl3, sWr0, sWr1, sWr2, sWr3, Wlin, blin):
    Ws = [W0, W1, W2, W3]
    bs = [b0, b1, b2, b3]
    sWls = [sWl0, sWl1, sWl2, sWl3]
    sbls = [sbl0, sbl1, sbl2, sbl3]
    sWrs = [sWr0, sWr1, sWr2, sWr3]

    n = x.shape[0]
    src = jnp.pad(edge_index[0], (0, EP - E))
    dst = jnp.pad(edge_index[1], (0, EP - E), constant_values=SENT)

    n_pad, chunks, C = _layer_dims(n)
    g = jnp.pad(x, ((0, n_pad - n), (0, 0)))
    t = None

    degsum = _deg2(n, n_pad, dst).reshape(n_pad, 1)

    for i in range(4):
        u, v = _tc_uv(g, t, Ws[i], bs[i], degsum)
        S = _make_rowagg(n, C, chunks)(u, src, dst)
        Wlr = jnp.concatenate([sWls[i], sWrs[i]], axis=1)
        xx, y2 = _tc_xy(S, v, degsum, Wlr, n_pad)

        yl = y2[:, 0]
        sagg = _edge_scalar_score(n, n_pad, src, dst, yl)
        score = sagg[:n] + sbls[i][0] + y2[:n, 1]

        k = int(math.ceil(0.5 * n))
        topv, perm = lax.top_k(score, k)
        tq = jnp.tanh(topv)

        k_pad_g = _rup(k, 4096)
        perm_g = jnp.pad(perm, (0, k_pad_g - k))
        g_full = _make_rowgather(k_pad_g)(xx, perm_g)

        k_s = _rup(k, 256)
        if i < 3:
            n_m = _rup(n + 2, 16 * 2048)
            k_pad_m = _rup(k, 2048)
            perm_m = jnp.pad(perm, (0, k_pad_m - k), constant_values=n_m - 1)
            mapping = _make_mapping(n, k, k_pad_m)(perm_m)
            nsrc, ndst = _make_relabel2(n, n_m)(src, dst, mapping)
            src, dst = nsrc, ndst
            degsum = _deg2(k, k_s, dst).reshape(k_s, 1)

        n = k
        n_pad, chunks, C = _layer_dims(n)
        g = g_full[:n_pad]
        t = jnp.pad(tq.reshape(k, 1), ((0, n_pad - n), (0, 0)))

    return _tc_final(g[:n], t[:n], Wlin, blin)
